# Initial kernel scaffold; baseline (speedup 1.0000x reference)
import jax
import jax.numpy as jnp
from jax.experimental import pallas as pl


def kernel(scores, deltas, anchors):
    raise NotImplementedError("write your pallas kernel here")


# speedup vs baseline: 10.9864x; 10.9864x over previous
import functools
import jax
import jax.numpy as jnp
from jax import lax
from jax.experimental import pallas as pl
from jax.experimental.pallas import tpu as pltpu
import numpy as np

_N = 20000
_NP = 20480
_PRE_K = 2000
_KP = 2048
_POST_K = 1000
_QP = 1024
_THR = 0.7
_IMG_H = 1024.0
_IMG_W = 1024.0
_CLAMP = float(np.log(1000.0 / 16.0))
_NEG = -1e9


def _decode_body(d_ref, a_ref, s_ref, bo_ref, so_ref):
    dx = d_ref[0]
    dy = d_ref[1]
    dw = jnp.minimum(d_ref[2], _CLAMP)
    dh = jnp.minimum(d_ref[3], _CLAMP)
    a0 = a_ref[0]
    a1 = a_ref[1]
    a2 = a_ref[2]
    a3 = a_ref[3]
    wa = a2 - a0
    ha = a3 - a1
    cxa = a0 + 0.5 * wa
    cya = a1 + 0.5 * ha
    px = dx * wa + cxa
    py = dy * ha + cya
    pw = jnp.exp(dw) * wa
    ph = jnp.exp(dh) * ha
    x1 = jnp.clip(px - 0.5 * pw, 0.0, _IMG_W)
    y1 = jnp.clip(py - 0.5 * ph, 0.0, _IMG_H)
    x2 = jnp.clip(px + 0.5 * pw, 0.0, _IMG_W)
    y2 = jnp.clip(py + 0.5 * ph, 0.0, _IMG_H)
    bo_ref[0] = x1
    bo_ref[1] = y1
    bo_ref[2] = x2
    bo_ref[3] = y2
    valid = ((x2 - x1) > 0.0) & ((y2 - y1) > 0.0)
    so_ref[...] = jnp.where(valid, s_ref[...], _NEG)


def _decode(d_t, a_t, s_r):
    return pl.pallas_call(
        _decode_body,
        out_shape=[
            jax.ShapeDtypeStruct((4, _NP // 128, 128), jnp.float32),
            jax.ShapeDtypeStruct((_NP // 128, 128), jnp.float32),
        ],
    )(d_t, a_t, s_r)


def _nms_body(bc_ref, br_ref, ts_ref, tsc_ref, out_ref,
              over_ref, keep_ref, kcol_ref, pos_ref, b_ref):
    f32 = jnp.float32
    x1c = bc_ref[0:1, :]
    y1c = bc_ref[1:2, :]
    x2c = bc_ref[2:3, :]
    y2c = bc_ref[3:4, :]
    area_c = (x2c - x1c) * (y2c - y1c)

    def iou_block(b, _):
        r0 = b * 128
        x1r = br_ref[0, pl.ds(r0, 128), :]
        y1r = br_ref[1, pl.ds(r0, 128), :]
        x2r = br_ref[2, pl.ds(r0, 128), :]
        y2r = br_ref[3, pl.ds(r0, 128), :]
        area_r = (x2r - x1r) * (y2r - y1r)
        w = jnp.maximum(jnp.minimum(x2r, x2c) - jnp.maximum(x1r, x1c), 0.0)
        h = jnp.maximum(jnp.minimum(y2r, y2c) - jnp.maximum(y1r, y1c), 0.0)
        inter = w * h
        iou = inter / (area_r + area_c - inter + 1e-9)
        over_ref[pl.ds(r0, 128), :] = (iou > _THR).astype(f32)
        return 0

    lax.fori_loop(0, _KP // 128, iou_block, 0)

    keep_ref[...] = jnp.ones((1, _KP), f32)
    cid = lax.broadcasted_iota(jnp.int32, (1, _KP), 1)

    def nms_step(g, _):
        keep = keep_ref[...]
        keep_g = jnp.sum(jnp.where(cid == g, keep, 0.0))
        row = over_ref[pl.ds(g, 1), :]
        later = (cid > g).astype(f32)
        sup = row * later * keep_g
        keep_ref[...] = keep * (1.0 - sup)
        return 0

    lax.fori_loop(0, _PRE_K, nms_step, 0)

    keep = keep_ref[...]
    ks = jnp.where(keep > 0.5, ts_ref[...], _NEG)

    def eye_block(b, _):
        r0 = b * 128
        ri = lax.broadcasted_iota(jnp.int32, (128, _KP), 0) + r0
        ci = lax.broadcasted_iota(jnp.int32, (128, _KP), 1)
        over_ref[pl.ds(r0, 128), :] = (ri == ci).astype(f32)
        return 0

    lax.fori_loop(0, _KP // 128, eye_block, 0)
    eye = over_ref[...]
    kcol_ref[...] = lax.dot_general(
        eye, keep, (((1,), (1,)), ((), ())),
        preferred_element_type=f32,
        precision=lax.Precision.HIGHEST)

    def rank_jc(jc, _):
        j0 = jc * 128
        kj = jnp.where(kcol_ref[pl.ds(j0, 128), :] > 0.5,
                       tsc_ref[pl.ds(j0, 128), :], _NEG)
        jidx = lax.broadcasted_iota(jnp.int32, (128, 1), 0) + j0

        def rank_kc(kc, acc):
            k0 = kc * 128
            kk = jnp.where(keep_ref[:, pl.ds(k0, 128)] > 0.5,
                           ts_ref[:, pl.ds(k0, 128)], _NEG)
            kidx = lax.broadcasted_iota(jnp.int32, (1, 128), 1) + k0
            gt = (kk > kj).astype(f32)
            eq = ((kk == kj) & (kidx < jidx)).astype(f32)
            return acc + jnp.sum(gt + eq, axis=1, keepdims=True)

        acc = lax.fori_loop(0, _KP // 128, rank_kc, jnp.zeros((128, 1), f32))
        pos_ref[pl.ds(j0, 128), :] = acc
        return 0

    lax.fori_loop(0, _KP // 128, rank_jc, 0)

    qid = lax.broadcasted_iota(jnp.int32, (1, _QP), 1).astype(f32)

    def onehot_jc(jc, _):
        j0 = jc * 128
        p = pos_ref[pl.ds(j0, 128), :]
        b_ref[pl.ds(j0, 128), :] = (p == qid).astype(f32)
        return 0

    lax.fori_loop(0, _KP // 128, onehot_jc, 0)
    data = jnp.concatenate(
        [bc_ref[...], ks, jnp.zeros((3, _KP), f32)], axis=0)
    out_ref[...] = lax.dot_general(
        data, b_ref[...], (((1,), (0,)), ((), ())),
        preferred_element_type=f32,
        precision=lax.Precision.HIGHEST)


def _nms_order(bc, br, ts_row, ts_col):
    f32 = jnp.float32
    return pl.pallas_call(
        _nms_body,
        out_shape=jax.ShapeDtypeStruct((8, _QP), f32),
        scratch_shapes=[
            pltpu.VMEM((_KP, _KP), f32),
            pltpu.VMEM((1, _KP), f32),
            pltpu.VMEM((_KP, 1), f32),
            pltpu.VMEM((_KP, 1), f32),
            pltpu.VMEM((_KP, _QP), f32),
        ],
    )(bc, br, ts_row, ts_col)


@jax.jit
def kernel(scores, deltas, anchors):
    f32 = jnp.float32
    pad = _NP - _N
    s_p = jnp.pad(scores, (0, pad))
    d_t = jnp.pad(deltas, ((0, pad), (0, 0))).T.reshape(4, _NP // 128, 128)
    a_t = jnp.pad(anchors, ((0, pad), (0, 0))).T.reshape(4, _NP // 128, 128)
    s_r = s_p.reshape(_NP // 128, 128)

    boxes, s_m = _decode(d_t, a_t, s_r)
    coords = boxes.reshape(4, _NP)[:, :_N]
    s_flat = s_m.reshape(_NP)[:_N]

    top_s, top_i = lax.top_k(s_flat, _PRE_K)
    tb = coords[:, top_i]

    kpad = _KP - _PRE_K
    bc = jnp.pad(tb, ((0, 0), (0, kpad)))
    ts = jnp.pad(top_s, (0, kpad), constant_values=_NEG)
    br = bc.reshape(4, 1, _KP).transpose(0, 2, 1)
    ts_row = ts.reshape(1, _KP)
    ts_col = ts.reshape(_KP, 1)

    out_t = _nms_order(bc, br, ts_row, ts_col)
    return out_t[:5, :_POST_K].T


# speedup vs baseline: 11.9146x; 1.0845x over previous
import functools
import jax
import jax.numpy as jnp
from jax import lax
from jax.experimental import pallas as pl
from jax.experimental.pallas import tpu as pltpu
import numpy as np

_N = 20000
_NP = 20480
_PRE_K = 2000
_KP = 2048
_POST_K = 1000
_QP = 1024
_THR = 0.7
_IMG_H = 1024.0
_IMG_W = 1024.0
_CLAMP = float(np.log(1000.0 / 16.0))
_NEG = -1e9


def _decode_body(d_ref, a_ref, s_ref, bo_ref, so_ref):
    dx = d_ref[0]
    dy = d_ref[1]
    dw = jnp.minimum(d_ref[2], _CLAMP)
    dh = jnp.minimum(d_ref[3], _CLAMP)
    a0 = a_ref[0]
    a1 = a_ref[1]
    a2 = a_ref[2]
    a3 = a_ref[3]
    wa = a2 - a0
    ha = a3 - a1
    cxa = a0 + 0.5 * wa
    cya = a1 + 0.5 * ha
    px = dx * wa + cxa
    py = dy * ha + cya
    pw = jnp.exp(dw) * wa
    ph = jnp.exp(dh) * ha
    x1 = jnp.clip(px - 0.5 * pw, 0.0, _IMG_W)
    y1 = jnp.clip(py - 0.5 * ph, 0.0, _IMG_H)
    x2 = jnp.clip(px + 0.5 * pw, 0.0, _IMG_W)
    y2 = jnp.clip(py + 0.5 * ph, 0.0, _IMG_H)
    bo_ref[0] = x1
    bo_ref[1] = y1
    bo_ref[2] = x2
    bo_ref[3] = y2
    valid = ((x2 - x1) > 0.0) & ((y2 - y1) > 0.0)
    so_ref[...] = jnp.where(valid, s_ref[...], _NEG)


def _decode(d_t, a_t, s_r):
    return pl.pallas_call(
        _decode_body,
        out_shape=[
            jax.ShapeDtypeStruct((4, _NP // 128, 128), jnp.float32),
            jax.ShapeDtypeStruct((_NP // 128, 128), jnp.float32),
        ],
    )(d_t, a_t, s_r)


def _nms_body(bc_ref, br_ref, ts_ref, tsc_ref, out_ref,
              over_ref, keep_ref, kcol_ref, pos_ref, b_ref, local_ref):
    f32 = jnp.float32
    x1c = bc_ref[0:1, :]
    y1c = bc_ref[1:2, :]
    x2c = bc_ref[2:3, :]
    y2c = bc_ref[3:4, :]
    area_c = (x2c - x1c) * (y2c - y1c)

    def iou_block(b, _):
        r0 = b * 128
        x1r = br_ref[0, pl.ds(r0, 128), :]
        y1r = br_ref[1, pl.ds(r0, 128), :]
        x2r = br_ref[2, pl.ds(r0, 128), :]
        y2r = br_ref[3, pl.ds(r0, 128), :]
        area_r = (x2r - x1r) * (y2r - y1r)
        w = jnp.maximum(jnp.minimum(x2r, x2c) - jnp.maximum(x1r, x1c), 0.0)
        h = jnp.maximum(jnp.minimum(y2r, y2c) - jnp.maximum(y1r, y1c), 0.0)
        inter = w * h
        iou = inter / (area_r + area_c - inter + 1e-9)
        over_ref[pl.ds(r0, 128), :] = (iou > _THR).astype(f32)
        return 0

    lax.fori_loop(0, _KP // 128, iou_block, 0)

    keep_ref[...] = jnp.ones((1, _KP), f32)
    cid = lax.broadcasted_iota(jnp.int32, (1, _KP), 1)
    lid = lax.broadcasted_iota(jnp.int32, (1, 128), 1)
    tri = (lax.broadcasted_iota(jnp.int32, (128, 128), 0) <
           lax.broadcasted_iota(jnp.int32, (128, 128), 1)).astype(f32)

    for b in range(_KP // 128):
        r0 = b * 128
        local_ref[...] = over_ref[r0:r0 + 128, r0:r0 + 128] * tri
        keepb0 = keep_ref[:, r0:r0 + 128]

        def nms_step(i, keepb):
            row = local_ref[pl.ds(i, 1), :]
            ki = jnp.sum(jnp.where(lid == i, keepb, 0.0))
            return keepb * (1.0 - row * ki)

        keepb = lax.fori_loop(0, 128, nms_step, keepb0)
        keep_ref[:, r0:r0 + 128] = keepb
        if b + 1 < _KP // 128:
            counts = lax.dot_general(
                keepb, over_ref[r0:r0 + 128, :],
                (((1,), (0,)), ((), ())),
                preferred_element_type=f32,
                precision=lax.Precision.HIGHEST)
            sup = ((counts > 0.5) & (cid >= r0 + 128)).astype(f32)
            keep_ref[...] = keep_ref[...] * (1.0 - sup)

    keep = keep_ref[...]
    ks = jnp.where(keep > 0.5, ts_ref[...], _NEG)

    def eye_block(b, _):
        r0 = b * 128
        ri = lax.broadcasted_iota(jnp.int32, (128, _KP), 0) + r0
        ci = lax.broadcasted_iota(jnp.int32, (128, _KP), 1)
        over_ref[pl.ds(r0, 128), :] = (ri == ci).astype(f32)
        return 0

    lax.fori_loop(0, _KP // 128, eye_block, 0)
    eye = over_ref[...]
    kcol_ref[...] = lax.dot_general(
        eye, keep, (((1,), (1,)), ((), ())),
        preferred_element_type=f32,
        precision=lax.Precision.HIGHEST)

    def rank_jc(jc, _):
        j0 = jc * 128
        kj = jnp.where(kcol_ref[pl.ds(j0, 128), :] > 0.5,
                       tsc_ref[pl.ds(j0, 128), :], _NEG)
        jidx = lax.broadcasted_iota(jnp.int32, (128, 1), 0) + j0

        def rank_kc(kc, acc):
            k0 = kc * 128
            kk = jnp.where(keep_ref[:, pl.ds(k0, 128)] > 0.5,
                           ts_ref[:, pl.ds(k0, 128)], _NEG)
            kidx = lax.broadcasted_iota(jnp.int32, (1, 128), 1) + k0
            gt = (kk > kj).astype(f32)
            eq = ((kk == kj) & (kidx < jidx)).astype(f32)
            return acc + jnp.sum(gt + eq, axis=1, keepdims=True)

        acc = lax.fori_loop(0, _KP // 128, rank_kc, jnp.zeros((128, 1), f32))
        pos_ref[pl.ds(j0, 128), :] = acc
        return 0

    lax.fori_loop(0, _KP // 128, rank_jc, 0)

    qid = lax.broadcasted_iota(jnp.int32, (1, _QP), 1).astype(f32)

    def onehot_jc(jc, _):
        j0 = jc * 128
        p = pos_ref[pl.ds(j0, 128), :]
        b_ref[pl.ds(j0, 128), :] = (p == qid).astype(f32)
        return 0

    lax.fori_loop(0, _KP // 128, onehot_jc, 0)
    data = jnp.concatenate(
        [bc_ref[...], ks, jnp.zeros((3, _KP), f32)], axis=0)
    out_ref[...] = lax.dot_general(
        data, b_ref[...], (((1,), (0,)), ((), ())),
        preferred_element_type=f32,
        precision=lax.Precision.HIGHEST)


def _nms_order(bc, br, ts_row, ts_col):
    f32 = jnp.float32
    return pl.pallas_call(
        _nms_body,
        out_shape=jax.ShapeDtypeStruct((8, _QP), f32),
        scratch_shapes=[
            pltpu.VMEM((_KP, _KP), f32),
            pltpu.VMEM((1, _KP), f32),
            pltpu.VMEM((_KP, 1), f32),
            pltpu.VMEM((_KP, 1), f32),
            pltpu.VMEM((_KP, _QP), f32),
            pltpu.VMEM((128, 128), f32),
        ],
    )(bc, br, ts_row, ts_col)


@jax.jit
def kernel(scores, deltas, anchors):
    f32 = jnp.float32
    pad = _NP - _N
    s_p = jnp.pad(scores, (0, pad))
    d_t = jnp.pad(deltas, ((0, pad), (0, 0))).T.reshape(4, _NP // 128, 128)
    a_t = jnp.pad(anchors, ((0, pad), (0, 0))).T.reshape(4, _NP // 128, 128)
    s_r = s_p.reshape(_NP // 128, 128)

    boxes, s_m = _decode(d_t, a_t, s_r)
    coords = boxes.reshape(4, _NP)[:, :_N]
    s_flat = s_m.reshape(_NP)[:_N]

    top_s, top_i = lax.top_k(s_flat, _PRE_K)
    tb = coords[:, top_i]

    kpad = _KP - _PRE_K
    bc = jnp.pad(tb, ((0, 0), (0, kpad)))
    ts = jnp.pad(top_s, (0, kpad), constant_values=_NEG)
    br = bc.reshape(4, 1, _KP).transpose(0, 2, 1)
    ts_row = ts.reshape(1, _KP)
    ts_col = ts.reshape(_KP, 1)

    out_t = _nms_order(bc, br, ts_row, ts_col)
    return out_t[:5, :_POST_K].T
